# baseline (speedup 1.0000x reference)
import jax
import jax.numpy as jnp
from jax import lax
from jax.experimental import pallas as pl
from jax.experimental.pallas import tpu as pltpu

NUM_KNOTS = 20
X_MIN = -5.0
X_MAX = 5.0
_DX = (X_MAX - X_MIN) / (NUM_KNOTS - 1)

_TBR = 256
_TBC = 2048


def _tc_body(x_ref, vals_ref, o_ref):
    xv = x_ref[...]
    xc = jnp.minimum(jnp.maximum(xv, jnp.float32(X_MIN)), jnp.float32(X_MAX))
    u = (xc - jnp.float32(X_MIN)) * jnp.float32(1.0 / _DX)
    i = u.astype(jnp.int32)
    f = u - i.astype(jnp.float32)

    acc3 = jnp.zeros_like(xv)
    acc2 = jnp.zeros_like(xv)
    acc1 = jnp.zeros_like(xv)
    acc0 = jnp.zeros_like(xv)
    for k in range(NUM_KNOTS):
        v0 = vals_ref[0, max(k - 1, 0)]
        v1 = vals_ref[0, k]
        v2 = vals_ref[0, min(k + 1, NUM_KNOTS - 1)]
        v3 = vals_ref[0, min(k + 2, NUM_KNOTS - 1)]
        c3k = 0.5 * (-v0 + 3.0 * v1 - 3.0 * v2 + v3)
        c2k = 0.5 * (2.0 * v0 - 5.0 * v1 + 4.0 * v2 - v3)
        c1k = 0.5 * (v2 - v0)
        c0k = v1
        m = i == k
        acc3 = jnp.where(m, c3k, acc3)
        acc2 = jnp.where(m, c2k, acc2)
        acc1 = jnp.where(m, c1k, acc1)
        acc0 = jnp.where(m, c0k, acc0)
    o_ref[...] = ((acc3 * f + acc2) * f + acc1) * f + acc0


def _tc_call(x, vals_pad):
    R, C = x.shape
    return pl.pallas_call(
        _tc_body,
        out_shape=jax.ShapeDtypeStruct((R, C), jnp.float32),
        grid=(R // _TBR, C // _TBC),
        in_specs=[
            pl.BlockSpec((_TBR, _TBC), lambda g, h: (g, h)),
            pl.BlockSpec(memory_space=pltpu.SMEM),
        ],
        out_specs=pl.BlockSpec((_TBR, _TBC), lambda g, h: (g, h)),
    )(x, vals_pad)


def kernel(x, values):
    vals_pad = jnp.pad(values, (0, 32 - NUM_KNOTS)).reshape(1, 32)
    return _tc_call(x, vals_pad)


# speedup vs baseline: 1.9415x; 1.9415x over previous
import functools

import jax
import jax.numpy as jnp
from jax import lax
from jax.experimental import pallas as pl
from jax.experimental.pallas import tpu as pltpu
from jax.experimental.pallas import tpu_sc as plsc

NUM_KNOTS = 20
X_MIN = -5.0
X_MAX = 5.0
_DX = (X_MAX - X_MIN) / (NUM_KNOTS - 1)

_NC = 2
_NS = 16
_NW = _NC * _NS
_L = 16

_BR = 2
_BC = 8192
_UNROLL = 2

_R_TC = 1280

_TBR = 256
_TBC = 2048


def _spline_body(x_hbm, values_hbm, out_hbm, vals_v, t3, t2, t1, t0,
                 xb0, xb1, ob0, ob1, si0, si1, so0, so1):
    wid = lax.axis_index("s") * _NC + lax.axis_index("c")
    n_rows, n_cols = out_hbm.shape
    row_skip = x_hbm.shape[0] - n_rows
    cblocks = n_cols // _BC
    rblocks_per_w = n_rows // (_BR * _NW)
    nblk = rblocks_per_w * cblocks
    row_base = wid * rblocks_per_w * _BR

    pltpu.sync_copy(values_hbm, vals_v)
    for t in range(2):
        k = lax.iota(jnp.int32, _L) + (t * _L)
        kc = jnp.minimum(k, NUM_KNOTS - 1)
        km1 = jnp.clip(k - 1, 0, NUM_KNOTS - 1)
        kp1 = jnp.minimum(k + 1, NUM_KNOTS - 1)
        kp2 = jnp.minimum(k + 2, NUM_KNOTS - 1)
        v0 = plsc.load_gather(vals_v, [km1])
        v1 = plsc.load_gather(vals_v, [kc])
        v2 = plsc.load_gather(vals_v, [kp1])
        v3 = plsc.load_gather(vals_v, [kp2])
        t3[pl.ds(t * _L, _L)] = 0.5 * (-v0 + 3.0 * v1 - 3.0 * v2 + v3)
        t2[pl.ds(t * _L, _L)] = 0.5 * (2.0 * v0 - 5.0 * v1 + 4.0 * v2 - v3)
        t1[pl.ds(t * _L, _L)] = 0.5 * (v2 - v0)
        t0[pl.ds(t * _L, _L)] = v1

    inv_dx = jnp.float32(1.0 / _DX)

    def blk_slice(b, skip):
        rb = b // cblocks
        cb = b - rb * cblocks
        return (pl.ds(skip + row_base + rb * _BR, _BR), pl.ds(cb * _BC, _BC))

    def compute_block(xb, ob):
        @plsc.parallel_loop(0, _BC // _L, 1, unroll=_UNROLL)
        def _vec_body(j):
            s = j * _L
            for r in range(_BR):
                xv = xb[r, pl.ds(s, _L)]
                xc = jnp.minimum(jnp.maximum(xv, jnp.float32(X_MIN)),
                                 jnp.float32(X_MAX))
                u = (xc - jnp.float32(X_MIN)) * inv_dx
                i = u.astype(jnp.int32)
                f = u - i.astype(jnp.float32)
                c3 = plsc.load_gather(t3, [i])
                c2 = plsc.load_gather(t2, [i])
                c1 = plsc.load_gather(t1, [i])
                c0 = plsc.load_gather(t0, [i])
                ob[r, pl.ds(s, _L)] = ((c3 * f + c2) * f + c1) * f + c0

    r0, c0_ = blk_slice(0, row_skip)
    pltpu.async_copy(x_hbm.at[r0, c0_], xb0, si0)
    r1, c1_ = blk_slice(1, row_skip)
    pltpu.async_copy(x_hbm.at[r1, c1_], xb1, si1)

    bufs = ((xb0, ob0, si0, so0), (xb1, ob1, si1, so1))

    def pair_body(g2, _):
        for b, (xb, ob, si, so) in enumerate(bufs):
            c = 2 * g2 + b
            xrs, xcs = blk_slice(c, row_skip)
            ors, ocs = blk_slice(c, 0)
            pltpu.make_async_copy(x_hbm.at[xrs, xcs], xb, si).wait()

            @pl.when(g2 > 0)
            def _wait_prev_out():
                pltpu.make_async_copy(ob, out_hbm.at[ors, ocs], so).wait()

            compute_block(xb, ob)
            pltpu.async_copy(ob, out_hbm.at[ors, ocs], so)

            @pl.when(c + 2 < nblk)
            def _start_next_in():
                xrs2, xcs2 = blk_slice(c + 2, row_skip)
                pltpu.async_copy(x_hbm.at[xrs2, xcs2], xb, si)
        return 0

    lax.fori_loop(0, nblk // 2, pair_body, 0)

    rs, cs = blk_slice(nblk - 2, 0)
    pltpu.make_async_copy(ob0, out_hbm.at[rs, cs], so0).wait()
    rs, cs = blk_slice(nblk - 1, 0)
    pltpu.make_async_copy(ob1, out_hbm.at[rs, cs], so1).wait()


def _sc_call(x, vals_pad32, n_rows_sc):
    mesh = plsc.VectorSubcoreMesh(core_axis_name="c", subcore_axis_name="s")
    run = functools.partial(
        pl.kernel,
        mesh=mesh,
        compiler_params=pltpu.CompilerParams(needs_layout_passes=False),
        out_type=jax.ShapeDtypeStruct((n_rows_sc, x.shape[1]), jnp.float32),
        scratch_types=[
            pltpu.VMEM((32,), jnp.float32),
            pltpu.VMEM((32,), jnp.float32),
            pltpu.VMEM((32,), jnp.float32),
            pltpu.VMEM((32,), jnp.float32),
            pltpu.VMEM((32,), jnp.float32),
            pltpu.VMEM((_BR, _BC), jnp.float32),
            pltpu.VMEM((_BR, _BC), jnp.float32),
            pltpu.VMEM((_BR, _BC), jnp.float32),
            pltpu.VMEM((_BR, _BC), jnp.float32),
            pltpu.SemaphoreType.DMA,
            pltpu.SemaphoreType.DMA,
            pltpu.SemaphoreType.DMA,
            pltpu.SemaphoreType.DMA,
        ],
    )(_spline_body)
    return run(x, vals_pad32)


def _tc_body(x_ref, vals_ref, o_ref):
    xv = x_ref[...]
    xc = jnp.minimum(jnp.maximum(xv, jnp.float32(X_MIN)), jnp.float32(X_MAX))
    u = (xc - jnp.float32(X_MIN)) * jnp.float32(1.0 / _DX)
    i = u.astype(jnp.int32)
    f = u - i.astype(jnp.float32)

    acc3 = jnp.zeros_like(xv)
    acc2 = jnp.zeros_like(xv)
    acc1 = jnp.zeros_like(xv)
    acc0 = jnp.zeros_like(xv)
    for k in range(NUM_KNOTS):
        v0 = vals_ref[0, max(k - 1, 0)]
        v1 = vals_ref[0, k]
        v2 = vals_ref[0, min(k + 1, NUM_KNOTS - 1)]
        v3 = vals_ref[0, min(k + 2, NUM_KNOTS - 1)]
        c3k = 0.5 * (-v0 + 3.0 * v1 - 3.0 * v2 + v3)
        c2k = 0.5 * (2.0 * v0 - 5.0 * v1 + 4.0 * v2 - v3)
        c1k = 0.5 * (v2 - v0)
        c0k = v1
        m = i == k
        acc3 = jnp.where(m, c3k, acc3)
        acc2 = jnp.where(m, c2k, acc2)
        acc1 = jnp.where(m, c1k, acc1)
        acc0 = jnp.where(m, c0k, acc0)
    o_ref[...] = ((acc3 * f + acc2) * f + acc1) * f + acc0


def _tc_call(x, vals_pad, n_rows_tc):
    R, C = n_rows_tc, x.shape[1]
    return pl.pallas_call(
        _tc_body,
        out_shape=jax.ShapeDtypeStruct((R, C), jnp.float32),
        grid=(R // _TBR, C // _TBC),
        in_specs=[
            pl.BlockSpec((_TBR, _TBC), lambda g, h: (g, h)),
            pl.BlockSpec(memory_space=pltpu.SMEM),
        ],
        out_specs=pl.BlockSpec((_TBR, _TBC), lambda g, h: (g, h)),
    )(x, vals_pad)


def kernel(x, values):
    vals32 = jnp.pad(values, (0, 32 - NUM_KNOTS))
    sc_out = _sc_call(x, vals32, x.shape[0] - _R_TC)
    tc_out = _tc_call(x, vals32.reshape(1, 32), _R_TC)
    return jnp.concatenate([tc_out, sc_out], axis=0)


# speedup vs baseline: 2.0104x; 1.0355x over previous
import functools

import jax
import jax.numpy as jnp
from jax import lax
from jax.experimental import pallas as pl
from jax.experimental.pallas import tpu as pltpu
from jax.experimental.pallas import tpu_sc as plsc

NUM_KNOTS = 20
X_MIN = -5.0
X_MAX = 5.0
_DX = (X_MAX - X_MIN) / (NUM_KNOTS - 1)

_NC = 2
_NS = 16
_NW = _NC * _NS
_L = 16

_BR = 1
_BC = 8192
_NBUF = 4
_UNROLL = 2


def _spline_body(x_hbm, values_hbm, out_hbm, vals_v, t3, t2, t1, t0,
                 *bufs_and_sems):
    xbs = bufs_and_sems[0:_NBUF]
    obs = bufs_and_sems[_NBUF:2 * _NBUF]
    sis = bufs_and_sems[2 * _NBUF:3 * _NBUF]
    sos = bufs_and_sems[3 * _NBUF:4 * _NBUF]

    wid = lax.axis_index("s") * _NC + lax.axis_index("c")
    n_rows, n_cols = x_hbm.shape
    cblocks = n_cols // _BC
    rblocks_per_w = n_rows // (_BR * _NW)
    nblk = rblocks_per_w * cblocks
    row_base = wid * rblocks_per_w * _BR

    pltpu.sync_copy(values_hbm, vals_v)
    for t in range(2):
        k = lax.iota(jnp.int32, _L) + (t * _L)
        kc = jnp.minimum(k, NUM_KNOTS - 1)
        km1 = jnp.clip(k - 1, 0, NUM_KNOTS - 1)
        kp1 = jnp.minimum(k + 1, NUM_KNOTS - 1)
        kp2 = jnp.minimum(k + 2, NUM_KNOTS - 1)
        v0 = plsc.load_gather(vals_v, [km1])
        v1 = plsc.load_gather(vals_v, [kc])
        v2 = plsc.load_gather(vals_v, [kp1])
        v3 = plsc.load_gather(vals_v, [kp2])
        t3[pl.ds(t * _L, _L)] = 0.5 * (-v0 + 3.0 * v1 - 3.0 * v2 + v3)
        t2[pl.ds(t * _L, _L)] = 0.5 * (2.0 * v0 - 5.0 * v1 + 4.0 * v2 - v3)
        t1[pl.ds(t * _L, _L)] = 0.5 * (v2 - v0)
        t0[pl.ds(t * _L, _L)] = v1

    inv_dx = jnp.float32(1.0 / _DX)

    def blk_slice(b):
        rb = b // cblocks
        cb = b - rb * cblocks
        return (pl.ds(row_base + rb * _BR, _BR), pl.ds(cb * _BC, _BC))

    def compute_block(xb, ob):
        @plsc.parallel_loop(0, _BC // _L, 1, unroll=_UNROLL)
        def _vec_body(j):
            s = j * _L
            for r in range(_BR):
                xv = xb[r, pl.ds(s, _L)]
                xc = jnp.minimum(jnp.maximum(xv, jnp.float32(X_MIN)),
                                 jnp.float32(X_MAX))
                u = (xc - jnp.float32(X_MIN)) * inv_dx
                i = u.astype(jnp.int32)
                f = u - i.astype(jnp.float32)
                c3 = plsc.load_gather(t3, [i])
                c2 = plsc.load_gather(t2, [i])
                c1 = plsc.load_gather(t1, [i])
                c0 = plsc.load_gather(t0, [i])
                ob[r, pl.ds(s, _L)] = ((c3 * f + c2) * f + c1) * f + c0

    for b in range(_NBUF):
        rs, cs = blk_slice(b)
        pltpu.async_copy(x_hbm.at[rs, cs], xbs[b], sis[b])

    def ring_body(g, _):
        for b in range(_NBUF):
            c = g * _NBUF + b
            rs, cs = blk_slice(c)
            pltpu.make_async_copy(x_hbm.at[rs, cs], xbs[b], sis[b]).wait()

            @pl.when(g > 0)
            def _wait_prev_out():
                pltpu.make_async_copy(
                    obs[b], out_hbm.at[rs, cs], sos[b]).wait()

            compute_block(xbs[b], obs[b])
            pltpu.async_copy(obs[b], out_hbm.at[rs, cs], sos[b])

            @pl.when(c + _NBUF < nblk)
            def _start_next_in():
                rs2, cs2 = blk_slice(c + _NBUF)
                pltpu.async_copy(x_hbm.at[rs2, cs2], xbs[b], sis[b])
        return 0

    lax.fori_loop(0, nblk // _NBUF, ring_body, 0)

    for b in range(_NBUF):
        rs, cs = blk_slice(nblk - _NBUF + b)
        pltpu.make_async_copy(obs[b], out_hbm.at[rs, cs], sos[b]).wait()


def kernel(x, values):
    mesh = plsc.VectorSubcoreMesh(core_axis_name="c", subcore_axis_name="s")
    vals_pad = jnp.pad(values, (0, 32 - NUM_KNOTS))
    scratch = [pltpu.VMEM((32,), jnp.float32)] * 5
    scratch += [pltpu.VMEM((_BR, _BC), jnp.float32)] * (2 * _NBUF)
    scratch += [pltpu.SemaphoreType.DMA] * (2 * _NBUF)
    run = functools.partial(
        pl.kernel,
        mesh=mesh,
        compiler_params=pltpu.CompilerParams(needs_layout_passes=False),
        out_type=jax.ShapeDtypeStruct(x.shape, jnp.float32),
        scratch_types=scratch,
    )(_spline_body)
    return run(x, vals_pad)


# speedup vs baseline: 2.0957x; 1.0424x over previous
import functools

import jax
import jax.numpy as jnp
from jax import lax
from jax.experimental import pallas as pl
from jax.experimental.pallas import tpu as pltpu
from jax.experimental.pallas import tpu_sc as plsc

NUM_KNOTS = 20
X_MIN = -5.0
X_MAX = 5.0
_DX = (X_MAX - X_MIN) / (NUM_KNOTS - 1)

_NC = 2
_NS = 16
_NW = _NC * _NS
_L = 16

_BR = 2
_BC = 8192
_NBUF = 4
_UNROLL = 2


def _spline_body(x_hbm, values_hbm, out_hbm, vals_v, t3, t2, t1, t0,
                 *bufs_and_sems):
    bufs = bufs_and_sems[0:_NBUF]
    sis = bufs_and_sems[_NBUF:2 * _NBUF]
    sos = bufs_and_sems[2 * _NBUF:3 * _NBUF]

    wid = lax.axis_index("s") * _NC + lax.axis_index("c")
    n_rows, n_cols = x_hbm.shape
    cblocks = n_cols // _BC
    rblocks_per_w = n_rows // (_BR * _NW)
    nblk = rblocks_per_w * cblocks
    row_base = wid * rblocks_per_w * _BR

    pltpu.sync_copy(values_hbm, vals_v)
    for t in range(2):
        k = lax.iota(jnp.int32, _L) + (t * _L)
        kc = jnp.minimum(k, NUM_KNOTS - 1)
        km1 = jnp.clip(k - 1, 0, NUM_KNOTS - 1)
        kp1 = jnp.minimum(k + 1, NUM_KNOTS - 1)
        kp2 = jnp.minimum(k + 2, NUM_KNOTS - 1)
        v0 = plsc.load_gather(vals_v, [km1])
        v1 = plsc.load_gather(vals_v, [kc])
        v2 = plsc.load_gather(vals_v, [kp1])
        v3 = plsc.load_gather(vals_v, [kp2])
        t3[pl.ds(t * _L, _L)] = 0.5 * (-v0 + 3.0 * v1 - 3.0 * v2 + v3)
        t2[pl.ds(t * _L, _L)] = 0.5 * (2.0 * v0 - 5.0 * v1 + 4.0 * v2 - v3)
        t1[pl.ds(t * _L, _L)] = 0.5 * (v2 - v0)
        t0[pl.ds(t * _L, _L)] = v1

    inv_dx = jnp.float32(1.0 / _DX)

    def blk_slice(b):
        rb = b // cblocks
        cb = b - rb * cblocks
        return (pl.ds(row_base + rb * _BR, _BR), pl.ds(cb * _BC, _BC))

    def compute_block(xb):
        @plsc.parallel_loop(0, _BC // _L, 1, unroll=_UNROLL)
        def _vec_body(j):
            s = j * _L
            for r in range(_BR):
                xv = xb[r, pl.ds(s, _L)]
                xc = jnp.minimum(jnp.maximum(xv, jnp.float32(X_MIN)),
                                 jnp.float32(X_MAX))
                u = (xc - jnp.float32(X_MIN)) * inv_dx
                i = u.astype(jnp.int32)
                f = u - i.astype(jnp.float32)
                c3 = plsc.load_gather(t3, [i])
                c2 = plsc.load_gather(t2, [i])
                c1 = plsc.load_gather(t1, [i])
                c0 = plsc.load_gather(t0, [i])
                xb[r, pl.ds(s, _L)] = ((c3 * f + c2) * f + c1) * f + c0

    for b in range(2):
        rs, cs = blk_slice(b)
        pltpu.async_copy(x_hbm.at[rs, cs], bufs[b], sis[b])

    def ring_body(g, _):
        for b in range(_NBUF):
            c = g * _NBUF + b
            rs, cs = blk_slice(c)
            pltpu.make_async_copy(x_hbm.at[rs, cs], bufs[b], sis[b]).wait()
            compute_block(bufs[b])
            pltpu.async_copy(bufs[b], out_hbm.at[rs, cs], sos[b])

            bm = (b + 2) % _NBUF

            @pl.when(c + 2 < nblk)
            def _start_ahead():
                @pl.when(c >= 2)
                def _wait_old_out():
                    rso, cso = blk_slice(c - 2)
                    pltpu.make_async_copy(
                        bufs[bm], out_hbm.at[rso, cso], sos[bm]).wait()
                rs2, cs2 = blk_slice(c + 2)
                pltpu.async_copy(x_hbm.at[rs2, cs2], bufs[bm], sis[bm])
        return 0

    lax.fori_loop(0, nblk // _NBUF, ring_body, 0)

    for b in range(_NBUF):
        c = nblk - _NBUF + b
        rs, cs = blk_slice(c)
        pltpu.make_async_copy(bufs[c % _NBUF], out_hbm.at[rs, cs],
                              sos[c % _NBUF]).wait()


def kernel(x, values):
    mesh = plsc.VectorSubcoreMesh(core_axis_name="c", subcore_axis_name="s")
    vals_pad = jnp.pad(values, (0, 32 - NUM_KNOTS))
    scratch = [pltpu.VMEM((32,), jnp.float32)] * 5
    scratch += [pltpu.VMEM((_BR, _BC), jnp.float32)] * _NBUF
    scratch += [pltpu.SemaphoreType.DMA] * (2 * _NBUF)
    run = functools.partial(
        pl.kernel,
        mesh=mesh,
        compiler_params=pltpu.CompilerParams(needs_layout_passes=False),
        out_type=jax.ShapeDtypeStruct(x.shape, jnp.float32),
        scratch_types=scratch,
    )(_spline_body)
    return run(x, vals_pad)
